# baseline (speedup 1.0000x reference)
import functools

import jax
import jax.numpy as jnp
from jax import lax
from jax.experimental import pallas as pl
from jax.experimental.pallas import tpu as pltpu
from jax.experimental.pallas import tpu_sc as plsc

B = 2
L = 4096
G = 2
DG = 16
K = 64
BQ = 512
NW = 32
QW = (B * L) // NW


NB = 256


def _onehot(byte_col, dtype=jnp.bfloat16):
    n = byte_col.shape[0]
    lane = lax.broadcasted_iota(jnp.int32, (n, NB), 1)
    return jnp.where(byte_col == lane, 1.0, 0.0).astype(dtype)


def _onehot_t(byte_row, height=NB):
    n = byte_row.shape[1]
    sub = lax.broadcasted_iota(jnp.int32, (height, n), 0)
    return jnp.where(byte_row == sub, 1.0, 0.0).astype(jnp.bfloat16)


def _dot(a, b):
    return lax.dot_general(a, b, (((1,), (0,)), ((), ())),
                           preferred_element_type=jnp.float32)


def _dot_hi(a, b):
    return lax.dot_general(a, b, (((1,), (0,)), ((), ())),
                           precision=lax.Precision.HIGHEST,
                           preferred_element_type=jnp.float32)


def _rowsum(x):
    ones = jnp.ones((x.shape[1], 1), jnp.float32)
    return _dot_hi(x, ones)


def _split_pick(ohh, wa, wb, ohl_f32):
    m = _dot(ohh, wa) * 128.0 + _dot(ohh, wb)
    return _rowsum(m * ohl_f32)


def _dotT0(a, b):
    return lax.dot_general(a, b, (((0,), (0,)), ((), ())),
                           preferred_element_type=jnp.float32)


def _sig2_col(x):
    lane = lax.broadcasted_iota(jnp.int32, x.shape, 1)
    wl = lax.bitcast_convert_type(((lane & (DG - 1)) + 127) << 23, jnp.float32)
    pos = x > 0
    g0 = jnp.sum(jnp.where(pos & (lane < DG), wl, 0.0), axis=1, keepdims=True)
    g1 = jnp.sum(jnp.where(pos & (lane >= DG), wl, 0.0), axis=1, keepdims=True)
    return g0.astype(jnp.int32), g1.astype(jnp.int32)


def _prep_body(q_ref, k_ref, kf_ref, enc0_ref, rank0_ref, enc1_ref, rank1_ref,
               wa0_s, wb0_s, wi0_s, cra0_s, crb0_s, run0_s,
               wa1_s, wb1_s, wi1_s, cra1_s, crb1_s, run1_s):
    i = pl.program_id(1)
    scr = ((wa0_s, wb0_s, wi0_s, cra0_s, crb0_s, run0_s),
           (wa1_s, wb1_s, wi1_s, cra1_s, crb1_s, run1_s))
    outs = ((enc0_ref, rank0_ref), (enc1_ref, rank1_ref))

    @pl.when(i == 0)
    def _build():
        sigs = _sig2_col(kf_ref[0])
        rowb = lax.broadcasted_iota(jnp.int32, (NB, NB), 0)
        colb = lax.broadcasted_iota(jnp.int32, (NB, NB), 1)
        tri_lt = jnp.where(rowb < colb, 1.0, 0.0).astype(jnp.bfloat16)
        tri_gt = jnp.where(colb < rowb, 1.0, 0.0).astype(jnp.float32)
        for g in range(G):
            wa_s, wb_s, wi_s, cra_s, crb_s, run_s = scr[g]
            sig_c = sigs[g]
            ohh = _onehot(sig_c >> 8)
            ohl = _onehot(sig_c & (NB - 1))
            h2 = _dotT0(ohh, ohl)
            h2i = jnp.round(h2).astype(jnp.int32)
            h2a = (h2i >> 7).astype(jnp.bfloat16)
            h2b = (h2i & 127).astype(jnp.bfloat16)
            t = _dot(h2a, tri_lt) * 128.0 + _dot(h2b, tri_lt)
            r = jnp.sum(h2, axis=1, keepdims=True)
            c = _dot_hi(tri_gt, r)
            w = jnp.round(c + t).astype(jnp.int32)
            wa_s[...] = (w >> 7).astype(jnp.bfloat16)
            wb_s[...] = (w & 127).astype(jnp.bfloat16)
            wi_s[...] = w
            cr = jnp.round(c + r).astype(jnp.int32)
            cra_s[...] = (cr >> 7).astype(jnp.bfloat16)
            crb_s[...] = (cr & 127).astype(jnp.bfloat16)
            run_s[...] = jnp.zeros_like(run_s)

    qsigs = _sig2_col(q_ref[0])
    ksigs = _sig2_col(k_ref[0])
    rowi = lax.broadcasted_iota(jnp.int32, (BQ, BQ), 0)
    coli = lax.broadcasted_iota(jnp.int32, (BQ, BQ), 1)

    for g in range(G):
        wa_s, wb_s, wi_s, cra_s, crb_s, run_s = scr[g]
        enc_ref, rank_ref = outs[g]

        qsig = qsigs[g]
        ohh_q = _onehot(qsig >> 8)
        lw_q = qsig & (NB - 1)
        ohl_q = _onehot(lw_q, jnp.float32)
        ohl1_q = _onehot(lw_q + 1, jnp.float32)
        mw_q = _dot(ohh_q, wa_s[...]) * 128.0 + _dot(ohh_q, wb_s[...])
        crf = _dot(ohh_q, cra_s[...]) * 128.0 + _dot(ohh_q, crb_s[...])
        lof = _rowsum(mw_q * ohl_q)
        lef = _rowsum(mw_q * ohl1_q)
        lef = jnp.where(lw_q == NB - 1, crf, lef)
        lo = jnp.round(lof).astype(jnp.int32)
        cnt = jnp.round(lef).astype(jnp.int32) - lo
        pad = K - jnp.minimum(cnt, K)
        enc = ((lo - pad + K) << 7) | pad

        ksig = ksigs[g]
        ksig_r = jnp.transpose(ksig)
        ohh_k = _onehot(ksig >> 8)
        lw_k = ksig & (NB - 1)
        ohl_kf = _onehot(lw_k, jnp.float32)
        ohl_k = ohl_kf.astype(jnp.bfloat16)
        runi = run_s[...]
        v = wi_s[...] + runi
        va = (v >> 7).astype(jnp.bfloat16)
        vb = (v & 127).astype(jnp.bfloat16)
        rank_v = _split_pick(ohh_k, va, vb, ohl_kf)
        eq_blk = jnp.where((ksig == ksig_r) & (coli < rowi),
                           1.0, 0.0).astype(jnp.bfloat16)
        tie_blk = _dot(eq_blk, jnp.ones((BQ, 1), jnp.bfloat16))
        rank = jnp.round(rank_v + tie_blk).astype(jnp.int32)

        ohh_kt = _onehot_t(ksig_r >> 8)
        run_s[...] = runi + jnp.round(_dot(ohh_kt, ohl_k)).astype(jnp.int32)

        enc_ref[...] = jnp.transpose(enc)[0]
        rank_ref[...] = jnp.transpose(rank)[0]


def _sc_body(enc0_hbm, krank0_hbm, enc1_hbm, krank1_hbm, out_hbm,
             rank0_v, rank1_v, tab0_v, tab1_v, e0_v, e1_v, out_v, sem):
    cid = lax.axis_index("c")
    sid = lax.axis_index("s")
    wid = sid * 2 + cid
    b = wid // (NW // B)
    qoff = wid * QW
    roff = qoff - b * L
    iota = lax.broadcasted_iota(jnp.int32, (16,), 0)

    copies = [
        pltpu.make_async_copy(krank0_hbm.at[pl.ds(b * L, L)], rank0_v, sem),
        pltpu.make_async_copy(krank1_hbm.at[pl.ds(b * L, L)], rank1_v, sem),
        pltpu.make_async_copy(enc0_hbm.at[pl.ds(b * L + roff, QW)], e0_v, sem),
        pltpu.make_async_copy(enc1_hbm.at[pl.ds(b * L + roff, QW)], e1_v, sem),
    ]
    for cp in copies:
        cp.start()
    for cp in copies:
        cp.wait()

    for tab, rank_v in ((tab0_v, rank0_v), (tab1_v, rank1_v)):

        @plsc.parallel_loop(0, L // 16, unroll=8)
        def build(jb):
            rv = rank_v[pl.ds(jb * 16, 16)]
            plsc.store_scatter(tab, [rv], jb * 16 + iota)

    def run_queries(i_lo, i_hi):
      @plsc.parallel_loop(i_lo, i_hi, unroll=2)
      def qloop(i):
        isp = jnp.zeros((16,), jnp.int32) + i

        def cands(tab, e_v, reverse):
            ev = plsc.load_gather(e_v, [isp])
            padv = ev & 127
            base = ev >> 7
            cs = []
            for c in range(4):
                s = iota + c * 16
                if reverse:
                    addr = base - 1 - s
                    valid = (63 - s) >= padv
                else:
                    addr = base + (s - K)
                    valid = s >= padv
                addr = jnp.clip(addr, 0, L - 1)
                v = plsc.load_gather(tab, [addr])
                cs.append(jnp.where(valid, v, -1))
            return cs

        a = cands(tab0_v, e0_v, False)
        rb = cands(tab1_v, e1_v, True)
        l0 = jnp.minimum(a[0], rb[0])
        l1 = jnp.minimum(a[1], rb[1])
        l2 = jnp.minimum(a[2], rb[2])
        l3 = jnp.minimum(a[3], rb[3])
        m0 = jnp.minimum(l0, l2)
        m2 = jnp.maximum(l0, l2)
        m1 = jnp.minimum(l1, l3)
        m3 = jnp.maximum(l1, l3)
        n0 = jnp.minimum(m0, m1)
        n1 = jnp.maximum(m0, m1)
        n2 = jnp.minimum(m2, m3)
        n3 = jnp.maximum(m2, m3)
        for c, v in enumerate((n0, n1, n2, n3)):
            out_v[pl.ds(i * K + c * 16, 16)] = jnp.sort(v)

    half = QW * K // 2
    run_queries(0, QW // 2)
    cp_out = pltpu.make_async_copy(
        out_v.at[pl.ds(0, half)], out_hbm.at[pl.ds(qoff * K, half)], sem)
    cp_out.start()
    run_queries(QW // 2, QW)
    pltpu.sync_copy(out_v.at[pl.ds(half, half)],
                    out_hbm.at[pl.ds(qoff * K + half, half)])
    cp_out.wait()


def _run_prep(query_up, key_up, interpret=False):
    grid = (B, L // BQ)
    osd = jax.ShapeDtypeStruct((B * L,), jnp.int32)
    ospec = pl.BlockSpec((BQ,), lambda b, i: (b * (L // BQ) + i,))
    group_scratch = [
        pltpu.VMEM((NB, NB), jnp.bfloat16),
        pltpu.VMEM((NB, NB), jnp.bfloat16),
        pltpu.VMEM((NB, NB), jnp.int32),
        pltpu.VMEM((NB, 1), jnp.bfloat16),
        pltpu.VMEM((NB, 1), jnp.bfloat16),
        pltpu.VMEM((NB, NB), jnp.int32),
    ]
    return pl.pallas_call(
        _prep_body,
        grid=grid,
        in_specs=[
            pl.BlockSpec((1, BQ, G * DG), lambda b, i: (b, i, 0)),
            pl.BlockSpec((1, BQ, G * DG), lambda b, i: (b, i, 0)),
            pl.BlockSpec((1, L, G * DG), lambda b, i: (b, 0, 0)),
        ],
        out_specs=[ospec] * 4,
        out_shape=[osd] * 4,
        scratch_shapes=group_scratch * G,
        interpret=interpret,
    )(query_up, key_up, key_up)


def kernel(query_up, key_up, head_idx):
    del head_idx
    enc0, rank0, enc1, rank1 = _run_prep(query_up, key_up)

    mesh = plsc.VectorSubcoreMesh(core_axis_name="c", subcore_axis_name="s")
    sck = functools.partial(
        pl.kernel,
        mesh=mesh,
        compiler_params=pltpu.CompilerParams(needs_layout_passes=False),
        out_type=jax.ShapeDtypeStruct((B * L * K,), jnp.int32),
        scratch_types=[
            pltpu.VMEM((L,), jnp.int32),
            pltpu.VMEM((L,), jnp.int32),
            pltpu.VMEM((L,), jnp.int32),
            pltpu.VMEM((L,), jnp.int32),
            pltpu.VMEM((QW,), jnp.int32),
            pltpu.VMEM((QW,), jnp.int32),
            pltpu.VMEM((QW * K,), jnp.int32),
            pltpu.SemaphoreType.DMA,
        ],
    )(_sc_body)
    out = sck(enc0, rank0, enc1, rank1)
    return out.reshape(B, L, K)


# speedup vs baseline: 1.4562x; 1.4562x over previous
import functools

import jax
import jax.numpy as jnp
from jax import lax
from jax.experimental import pallas as pl
from jax.experimental.pallas import tpu as pltpu
from jax.experimental.pallas import tpu_sc as plsc

B = 2
L = 4096
G = 2
DG = 16
K = 64
BQ = 512
NW = 32
QW = (B * L) // NW


NB = 256


def _onehot(byte_col, dtype=jnp.bfloat16):
    n = byte_col.shape[0]
    lane = lax.broadcasted_iota(jnp.int32, (n, NB), 1)
    return jnp.where(byte_col == lane, 1.0, 0.0).astype(dtype)


def _onehot_t(byte_row, height=NB):
    n = byte_row.shape[1]
    sub = lax.broadcasted_iota(jnp.int32, (height, n), 0)
    return jnp.where(byte_row == sub, 1.0, 0.0).astype(jnp.bfloat16)


def _dot(a, b):
    return lax.dot_general(a, b, (((1,), (0,)), ((), ())),
                           preferred_element_type=jnp.float32)


def _dot_hi(a, b):
    return lax.dot_general(a, b, (((1,), (0,)), ((), ())),
                           precision=lax.Precision.HIGHEST,
                           preferred_element_type=jnp.float32)


def _split_pick(ohh, wa, wb, ohl_f32):
    m = _dot(ohh, wa) * 128.0 + _dot(ohh, wb)
    return jnp.sum(m * ohl_f32, axis=1, keepdims=True)


def _dotT0(a, b):
    return lax.dot_general(a, b, (((0,), (0,)), ((), ())),
                           preferred_element_type=jnp.float32)


def _sig2_col(x):
    lane = lax.broadcasted_iota(jnp.int32, x.shape, 1)
    wl = lax.bitcast_convert_type(((lane & (DG - 1)) + 127) << 23, jnp.float32)
    pos = x > 0
    g0 = jnp.sum(jnp.where(pos & (lane < DG), wl, 0.0), axis=1, keepdims=True)
    g1 = jnp.sum(jnp.where(pos & (lane >= DG), wl, 0.0), axis=1, keepdims=True)
    return g0.astype(jnp.int32), g1.astype(jnp.int32)


def _prep_body(q_ref, k_ref, kf_ref, enc0_ref, rank0_ref, enc1_ref, rank1_ref,
               wa0_s, wb0_s, wi0_s, cra0_s, crb0_s, run0_s,
               wa1_s, wb1_s, wi1_s, cra1_s, crb1_s, run1_s):
    i = pl.program_id(1)
    scr = ((wa0_s, wb0_s, wi0_s, cra0_s, crb0_s, run0_s),
           (wa1_s, wb1_s, wi1_s, cra1_s, crb1_s, run1_s))
    outs = ((enc0_ref, rank0_ref), (enc1_ref, rank1_ref))

    @pl.when(i == 0)
    def _build():
        sigs = _sig2_col(kf_ref[0])
        rowb = lax.broadcasted_iota(jnp.int32, (NB, NB), 0)
        colb = lax.broadcasted_iota(jnp.int32, (NB, NB), 1)
        tri_lt = jnp.where(rowb < colb, 1.0, 0.0).astype(jnp.bfloat16)
        tri_gt = jnp.where(colb < rowb, 1.0, 0.0).astype(jnp.float32)
        for g in range(G):
            wa_s, wb_s, wi_s, cra_s, crb_s, run_s = scr[g]
            sig_c = sigs[g]
            ohh = _onehot(sig_c >> 8)
            ohl = _onehot(sig_c & (NB - 1))
            h2 = _dotT0(ohh, ohl)
            h2i = jnp.round(h2).astype(jnp.int32)
            h2a = (h2i >> 7).astype(jnp.bfloat16)
            h2b = (h2i & 127).astype(jnp.bfloat16)
            t = _dot(h2a, tri_lt) * 128.0 + _dot(h2b, tri_lt)
            r = jnp.sum(h2, axis=1, keepdims=True)
            c = _dot_hi(tri_gt, r)
            w = jnp.round(c + t).astype(jnp.int32)
            wa_s[...] = (w >> 7).astype(jnp.bfloat16)
            wb_s[...] = (w & 127).astype(jnp.bfloat16)
            wi_s[...] = w
            cr = jnp.round(c + r).astype(jnp.int32)
            cra_s[...] = (cr >> 7).astype(jnp.bfloat16)
            crb_s[...] = (cr & 127).astype(jnp.bfloat16)
            run_s[...] = jnp.zeros_like(run_s)

    qsigs = _sig2_col(q_ref[0])
    ksigs = _sig2_col(k_ref[0])
    rowi = lax.broadcasted_iota(jnp.int32, (BQ, BQ), 0)
    coli = lax.broadcasted_iota(jnp.int32, (BQ, BQ), 1)

    for g in range(G):
        wa_s, wb_s, wi_s, cra_s, crb_s, run_s = scr[g]
        enc_ref, rank_ref = outs[g]

        qsig = qsigs[g]
        ohh_q = _onehot(qsig >> 8)
        lw_q = qsig & (NB - 1)
        ohl_q = _onehot(lw_q, jnp.float32)
        ohl1_q = _onehot(lw_q + 1, jnp.float32)
        mw_q = _dot(ohh_q, wa_s[...]) * 128.0 + _dot(ohh_q, wb_s[...])
        crf = _dot(ohh_q, cra_s[...]) * 128.0 + _dot(ohh_q, crb_s[...])
        lof = jnp.sum(mw_q * ohl_q, axis=1, keepdims=True)
        lef = jnp.sum(mw_q * ohl1_q, axis=1, keepdims=True)
        lef = jnp.where(lw_q == NB - 1, crf, lef)
        lo = jnp.round(lof).astype(jnp.int32)
        cnt = jnp.round(lef).astype(jnp.int32) - lo
        pad = K - jnp.minimum(cnt, K)
        enc = ((lo - pad + K) << 7) | pad

        ksig = ksigs[g]
        ksig_r = jnp.transpose(ksig)
        ohh_k = _onehot(ksig >> 8)
        lw_k = ksig & (NB - 1)
        ohl_kf = _onehot(lw_k, jnp.float32)
        ohl_k = ohl_kf.astype(jnp.bfloat16)
        runi = run_s[...]
        v = wi_s[...] + runi
        va = (v >> 7).astype(jnp.bfloat16)
        vb = (v & 127).astype(jnp.bfloat16)
        rank_v = _split_pick(ohh_k, va, vb, ohl_kf)
        eq_blk = (ksig == ksig_r) & (coli < rowi)
        tie_blk = jnp.sum(eq_blk.astype(jnp.int32), axis=1, keepdims=True)
        rank = jnp.round(rank_v).astype(jnp.int32) + tie_blk

        ohh_kt = _onehot_t(ksig_r >> 8)
        run_s[...] = runi + jnp.round(_dot(ohh_kt, ohl_k)).astype(jnp.int32)

        enc_ref[...] = jnp.transpose(enc)[0]
        rank_ref[...] = jnp.transpose(rank)[0]


def _sc_body(enc0_hbm, krank0_hbm, enc1_hbm, krank1_hbm, out_hbm,
             rank0_v, rank1_v, tab0_v, tab1_v, e0_v, e1_v, out_v, sem):
    cid = lax.axis_index("c")
    sid = lax.axis_index("s")
    wid = sid * 2 + cid
    b = wid // (NW // B)
    qoff = wid * QW
    roff = qoff - b * L
    iota = lax.broadcasted_iota(jnp.int32, (16,), 0)

    copies = [
        pltpu.make_async_copy(krank0_hbm.at[pl.ds(b * L, L)], rank0_v, sem),
        pltpu.make_async_copy(krank1_hbm.at[pl.ds(b * L, L)], rank1_v, sem),
        pltpu.make_async_copy(enc0_hbm.at[pl.ds(b * L + roff, QW)], e0_v, sem),
        pltpu.make_async_copy(enc1_hbm.at[pl.ds(b * L + roff, QW)], e1_v, sem),
    ]
    for cp in copies:
        cp.start()
    for cp in copies:
        cp.wait()

    for tab, rank_v in ((tab0_v, rank0_v), (tab1_v, rank1_v)):

        @plsc.parallel_loop(0, L // 16, unroll=8)
        def build(jb):
            rv = rank_v[pl.ds(jb * 16, 16)]
            plsc.store_scatter(tab, [rv], jb * 16 + iota)

    def run_queries(i_lo, i_hi):
      @plsc.parallel_loop(i_lo, i_hi, unroll=2)
      def qloop(i):
        isp = jnp.zeros((16,), jnp.int32) + i

        def cands(tab, e_v, reverse):
            ev = plsc.load_gather(e_v, [isp])
            padv = ev & 127
            base = ev >> 7
            cs = []
            for c in range(4):
                s = iota + c * 16
                if reverse:
                    addr = base - 1 - s
                    valid = (63 - s) >= padv
                else:
                    addr = base + (s - K)
                    valid = s >= padv
                addr = jnp.clip(addr, 0, L - 1)
                v = plsc.load_gather(tab, [addr])
                cs.append(jnp.where(valid, v, -1))
            return cs

        a = cands(tab0_v, e0_v, False)
        rb = cands(tab1_v, e1_v, True)
        l0 = jnp.minimum(a[0], rb[0])
        l1 = jnp.minimum(a[1], rb[1])
        l2 = jnp.minimum(a[2], rb[2])
        l3 = jnp.minimum(a[3], rb[3])
        m0 = jnp.minimum(l0, l2)
        m2 = jnp.maximum(l0, l2)
        m1 = jnp.minimum(l1, l3)
        m3 = jnp.maximum(l1, l3)
        n0 = jnp.minimum(m0, m1)
        n1 = jnp.maximum(m0, m1)
        n2 = jnp.minimum(m2, m3)
        n3 = jnp.maximum(m2, m3)
        for c, v in enumerate((n0, n1, n2, n3)):
            out_v[pl.ds(i * K + c * 16, 16)] = jnp.sort(v)

    half = QW * K // 2
    run_queries(0, QW // 2)
    cp_out = pltpu.make_async_copy(
        out_v.at[pl.ds(0, half)], out_hbm.at[pl.ds(qoff * K, half)], sem)
    cp_out.start()
    run_queries(QW // 2, QW)
    pltpu.sync_copy(out_v.at[pl.ds(half, half)],
                    out_hbm.at[pl.ds(qoff * K + half, half)])
    cp_out.wait()


def _run_prep(query_up, key_up, interpret=False):
    grid = (B, L // BQ)
    osd = jax.ShapeDtypeStruct((B * L,), jnp.int32)
    ospec = pl.BlockSpec((BQ,), lambda b, i: (b * (L // BQ) + i,))
    group_scratch = [
        pltpu.VMEM((NB, NB), jnp.bfloat16),
        pltpu.VMEM((NB, NB), jnp.bfloat16),
        pltpu.VMEM((NB, NB), jnp.int32),
        pltpu.VMEM((NB, 1), jnp.bfloat16),
        pltpu.VMEM((NB, 1), jnp.bfloat16),
        pltpu.VMEM((NB, NB), jnp.int32),
    ]
    return pl.pallas_call(
        _prep_body,
        grid=grid,
        in_specs=[
            pl.BlockSpec((1, BQ, G * DG), lambda b, i: (b, i, 0)),
            pl.BlockSpec((1, BQ, G * DG), lambda b, i: (b, i, 0)),
            pl.BlockSpec((1, L, G * DG), lambda b, i: (b, 0, 0)),
        ],
        out_specs=[ospec] * 4,
        out_shape=[osd] * 4,
        scratch_shapes=group_scratch * G,
        interpret=interpret,
    )(query_up, key_up, key_up)


def kernel(query_up, key_up, head_idx):
    del head_idx
    enc0, rank0, enc1, rank1 = _run_prep(query_up, key_up)

    mesh = plsc.VectorSubcoreMesh(core_axis_name="c", subcore_axis_name="s")
    sck = functools.partial(
        pl.kernel,
        mesh=mesh,
        compiler_params=pltpu.CompilerParams(needs_layout_passes=False),
        out_type=jax.ShapeDtypeStruct((B * L * K,), jnp.int32),
        scratch_types=[
            pltpu.VMEM((L,), jnp.int32),
            pltpu.VMEM((L,), jnp.int32),
            pltpu.VMEM((L,), jnp.int32),
            pltpu.VMEM((L,), jnp.int32),
            pltpu.VMEM((QW,), jnp.int32),
            pltpu.VMEM((QW,), jnp.int32),
            pltpu.VMEM((QW * K,), jnp.int32),
            pltpu.SemaphoreType.DMA,
        ],
    )(_sc_body)
    out = sck(enc0, rank0, enc1, rank1)
    return out.reshape(B, L, K)
